# baseline (speedup 1.0000x reference)
import jax
import jax.numpy as jnp
import numpy as np
from jax import lax
from jax.experimental import pallas as pl
from jax.experimental.pallas import tpu as pltpu
from jax.experimental.pallas import tpu_sc as plsc

B, R, N, F = 32, 1024, 64, 128
ROI_PER_IMAGE = 128
FG_NUM = 64
HARD_NUM = 51
EASY_NUM = 13
REG_FG_THRESH = 0.55
CLS_FG_THRESH = 0.75
CLS_BG_THRESH = 0.25
CLS_BG_THRESH_LO = 0.1

NVEC_R = R // 16
NVEC_S = ROI_PER_IMAGE // 16
NVEC_G = N // 16
RCLS_PAD = R + 64
GROWS = 80


def _host_perms() -> np.ndarray:
    u = np.uint32

    def tf(k, n):
        i = np.arange(n, dtype=np.uint64)
        x0 = (i >> np.uint64(32)).astype(u)
        x1 = (i & np.uint64(0xFFFFFFFF)).astype(u)
        rot = ((13, 15, 26, 6), (17, 29, 16, 24))
        ks = (u(k[0]), u(k[1]), u(k[0]) ^ u(k[1]) ^ u(0x1BD11BDA))
        sched = ((ks[1], ks[2]), (ks[2], ks[0]), (ks[0], ks[1]),
                 (ks[1], ks[2]), (ks[2], ks[0]))
        with np.errstate(over="ignore"):
            x0 = x0 + ks[0]
            x1 = x1 + ks[1]
            for gi in range(5):
                for rr in rot[gi % 2]:
                    x0 = x0 + x1
                    x1 = (x1 << u(rr)) | (x1 >> u(32 - rr))
                    x1 = x1 ^ x0
                x0 = x0 + sched[gi][0]
                x1 = x1 + sched[gi][1] + u(gi + 1)
        return x0, x1

    def split(k, n):
        x0, x1 = tf(k, n)
        return list(zip(x0, x1))

    def uniform01(k, n):
        x0, x1 = tf(k, n)
        bits = x0 ^ x1
        return ((bits >> u(9)) | u(0x3F800000)).view(np.float32) - np.float32(1.0)

    r = np.stack([np.stack([uniform01(kj, R) for kj in split(kb, 3)])
                  for kb in split((u(0), u(1)), B)])
    return np.argsort(-r, axis=-1, kind="stable").astype(np.int32)


_PERM = _host_perms().reshape(B, 3 * R)


def _body(rois_hbm, gtc_hbm, lab_hbm, sco_hbm, perm_hbm, feat_hbm,
          o_br, o_bgr, o_bi, o_bs, o_bl, o_bf, o_rvm, o_cls,
          v_rois, v_gtc, v_lab, v_sco, v_perm,
          v_bmin, v_bmax, v_vb, v_gorig, v_rcls,
          v_mo, v_asg, v_idx, v_gidx,
          v_br, v_bgr, v_bi, v_bs, v_bl, v_rvm, v_cls, v_bf,
          sem):
    b = lax.axis_index("c") * 16 + lax.axis_index("s")

    cp_in = [
        pltpu.async_copy(rois_hbm.at[:, b], v_rois, sem),
        pltpu.async_copy(gtc_hbm.at[b], v_gtc, sem),
        pltpu.async_copy(lab_hbm.at[b], v_lab, sem),
        pltpu.async_copy(sco_hbm.at[b], v_sco, sem),
        pltpu.async_copy(perm_hbm.at[b], v_perm, sem),
    ]
    for c in cp_in:
        c.wait()

    lane = lax.iota(jnp.int32, 16)
    zero16i = jnp.full((16,), 0, jnp.int32)
    qv = [zero16i + q for q in range(8)]

    glab = [v_gtc[7, pl.ds(ch * 16, 16)].astype(jnp.int32)
            for ch in range(NVEC_G)]
    gcnt = [jnp.full((16,), 0, jnp.int32) for _ in range(3)]
    for ch in range(NVEC_G):
        for c in range(3):
            gcnt[c] = gcnt[c] + plsc.all_reduce_population_count(glab[ch] == c + 1)
    a0 = (gcnt[0] + 3) & -4
    a1 = (gcnt[1] + 3) & -4
    gstart = [zero16i, a0, a0 + a1]

    grank = [jnp.full((16,), 0, jnp.int32) for _ in range(3)]
    for ch in range(NVEC_G):
        o = ch * 16
        labv = glab[ch]
        dest = zero16i
        for c in range(3):
            m = labv == c + 1
            pc = plsc.cumsum(m.astype(jnp.int32))
            dest = jnp.where(m, gstart[c] + grank[c] + pc - 1, dest)
            grank[c] = grank[c] + plsc.all_reduce_population_count(m)
        plsc.store_scatter(v_gorig, [dest], lane + o)
        c0 = v_gtc[0, pl.ds(o, 16)]
        c1 = v_gtc[1, pl.ds(o, 16)]
        c2 = v_gtc[2, pl.ds(o, 16)]
        s0 = v_gtc[3, pl.ds(o, 16)]
        s1 = v_gtc[4, pl.ds(o, 16)]
        s2 = v_gtc[5, pl.ds(o, 16)]
        rows = (c0 - s0 * 0.5, c1 - s1 * 0.5, c2 - s2 * 0.5,
                c0 + s0 * 0.5, c1 + s1 * 0.5, c2 + s2 * 0.5,
                (s0 * s1) * s2)
        refs = (v_bmin, v_bmin, v_bmin, v_bmax, v_bmax, v_bmax, v_vb)
        offs = (0, GROWS * 16, 2 * GROWS * 16, 0, GROWS * 16, 2 * GROWS * 16, 0)
        dest16 = dest * 16
        for l in range(16):
            idx = dest16 + l
            for rowv, ref, qoff in zip(rows, refs, offs):
                plsc.store_scatter(ref, [idx + qoff], rowv)

    def zero_rcls(v, _):
        v_rcls[pl.ds(v * 16, 16)] = zero16i
        return ()

    lax.fori_loop(0, RCLS_PAD // 16, zero_rcls, ())

    rbase = [zero16i]
    rcnt = []
    for c in range(3):
        base = rbase[c]

        def rpass(v, cnt, c=c, base=base):
            labv = v_lab[pl.ds(v * 16, 16)]
            m = labv == c + 1
            pc = plsc.cumsum(m.astype(jnp.int32))
            slot = base + cnt + pc - 1
            plsc.store_scatter(v_rcls, [slot], lane + v * 16, mask=m)
            return cnt + plsc.all_reduce_population_count(m)

        cnt = lax.fori_loop(0, NVEC_R, rpass, zero16i)
        rcnt.append(cnt)
        rbase.append(base + ((cnt + 15) & -16))

    for c in range(3):
        r_lo = jnp.max(rbase[c])
        r_n = jnp.max(rcnt[c])
        g_lo = jnp.max(gstart[c])
        g_hi = g_lo + jnp.max(gcnt[c])
        g_n4 = (g_hi - g_lo + 3) // 4
        nchunk = (r_n + 15) // 16

        def chunk_body(j, _, r_lo=r_lo, r_n=r_n, g_lo=g_lo, g_hi=g_hi,
                       g_n4=g_n4):
            idxv = v_rcls[pl.ds(r_lo + j * 16, 16)]
            lanemask = lane < (r_n - j * 16)
            cx = plsc.load_gather(v_rois, [qv[0], idxv])
            cy = plsc.load_gather(v_rois, [qv[1], idxv])
            cz = plsc.load_gather(v_rois, [qv[2], idxv])
            dx = plsc.load_gather(v_rois, [qv[3], idxv])
            dy = plsc.load_gather(v_rois, [qv[4], idxv])
            dz = plsc.load_gather(v_rois, [qv[5], idxv])
            hx = dx * 0.5
            hy = dy * 0.5
            hz = dz * 0.5
            ax0 = cx - hx
            ax1 = cx + hx
            ay0 = cy - hy
            ay1 = cy + hy
            az0 = cz - hz
            az1 = cz + hz
            va = dx * dy * dz

            ghiv = zero16i + g_hi

            def inner(i4, carry):
                bn, bd, bidx = carry
                g0 = g_lo + i4 * 4
                for rr in range(4):
                    g = g0 + rr
                    go = g * 16
                    ix = jnp.maximum(jnp.minimum(ax1, v_bmax[pl.ds(go, 16)])
                                     - jnp.maximum(ax0, v_bmin[pl.ds(go, 16)]), 0.0)
                    iy = jnp.maximum(jnp.minimum(ay1, v_bmax[pl.ds(GROWS * 16 + go, 16)])
                                     - jnp.maximum(ay0, v_bmin[pl.ds(GROWS * 16 + go, 16)]), 0.0)
                    iz = jnp.maximum(jnp.minimum(az1, v_bmax[pl.ds(2 * GROWS * 16 + go, 16)])
                                     - jnp.maximum(az0, v_bmin[pl.ds(2 * GROWS * 16 + go, 16)]), 0.0)
                    iv = (ix * iy) * iz
                    den = jnp.maximum(va + v_vb[pl.ds(go, 16)] - iv, 1e-6)
                    gvec = zero16i + g
                    better = jnp.logical_and(iv * bd > bn * den, gvec < ghiv)
                    bn = jnp.where(better, iv, bn)
                    bd = jnp.where(better, den, bd)
                    bidx = jnp.where(better, gvec, bidx)
                return bn, bd, bidx

            init = (jnp.full((16,), 0.0, jnp.float32),
                    jnp.full((16,), 1.0, jnp.float32),
                    jnp.full((16,), -1, jnp.int32))
            bn, bd, bidx = lax.fori_loop(0, g_n4, inner, init)
            mo = bn / bd
            asg = plsc.load_gather(v_gorig, [jnp.maximum(bidx, 0)])
            asg = jnp.where(bidx < 0, 0, asg)
            plsc.store_scatter(v_mo, [idxv], mo, mask=lanemask)
            plsc.store_scatter(v_asg, [idxv], asg, mask=lanemask)
            return ()

        lax.fori_loop(0, nchunk, chunk_body, ())

    def sample_pool(pool, off, k):
        thresholds = {
            0: lambda v: v >= REG_FG_THRESH,
            1: lambda v: jnp.logical_and(v < REG_FG_THRESH, v >= CLS_BG_THRESH_LO),
            2: lambda v: v < CLS_BG_THRESH_LO,
        }
        mask_fn = thresholds[pool]

        def pass_a(v, cnt):
            pidx = v_perm[pl.ds(pool * R + v * 16, 16)]
            vals = plsc.load_gather(v_mo, [pidx])
            m = mask_fn(vals)
            pc = plsc.cumsum(m.astype(jnp.int32))
            slot = cnt + pc - 1
            ok = jnp.logical_and(m, slot < k)
            plsc.store_scatter(v_idx, [slot + off], pidx, mask=ok)
            return cnt + plsc.all_reduce_population_count(m)

        cnt = lax.fori_loop(0, NVEC_R, pass_a, zero16i)

        def pass_b(v, cnt):
            pidx = lane + v * 16
            vals = v_mo[pl.ds(v * 16, 16)]
            m = jnp.logical_not(mask_fn(vals))
            pc = plsc.cumsum(m.astype(jnp.int32))
            slot = cnt + pc - 1
            ok = jnp.logical_and(m, slot < k)
            plsc.store_scatter(v_idx, [slot + off], pidx, mask=ok)
            return cnt + plsc.all_reduce_population_count(m)

        @pl.when(jnp.max(cnt) < k)
        def _():
            lax.fori_loop(0, NVEC_R, pass_b, cnt)

    sample_pool(0, 0, FG_NUM)
    sample_pool(1, FG_NUM, HARD_NUM)
    sample_pool(2, FG_NUM + HARD_NUM, EASY_NUM)

    for s in range(NVEC_S):
        sl = pl.ds(s * 16, 16)
        sidx = v_idx[sl]
        iou_s = plsc.load_gather(v_mo, [sidx])
        v_bi[sl] = iou_s
        v_rvm[sl] = (iou_s > REG_FG_THRESH).astype(jnp.int32)
        fgm = iou_s > CLS_FG_THRESH
        bgm = iou_s < CLS_BG_THRESH
        interval = jnp.logical_and(jnp.logical_not(fgm), jnp.logical_not(bgm))
        v_cls[sl] = jnp.where(interval, (iou_s - CLS_BG_THRESH) * 2.0,
                              jnp.where(fgm, 1.0, 0.0))
        v_bl[sl] = plsc.load_gather(v_lab, [sidx])
        v_bs[sl] = plsc.load_gather(v_sco, [sidx])
        asgs = plsc.load_gather(v_asg, [sidx])
        for dd in range(7):
            v_br[dd, sl] = plsc.load_gather(v_rois, [qv[dd], sidx])
            v_bgr[dd, sl] = plsc.load_gather(v_gtc, [qv[dd], asgs])
        v_gidx[sl] = sidx + b * R

    pltpu.async_copy(feat_hbm.at[v_gidx], v_bf, sem).wait()

    cp_out = [
        pltpu.async_copy(v_br, o_br.at[:, b], sem),
        pltpu.async_copy(v_bgr, o_bgr.at[:, b], sem),
        pltpu.async_copy(v_bi, o_bi.at[b], sem),
        pltpu.async_copy(v_bs, o_bs.at[b], sem),
        pltpu.async_copy(v_bl, o_bl.at[b], sem),
        pltpu.async_copy(v_bf, o_bf.at[b], sem),
        pltpu.async_copy(v_rvm, o_rvm.at[b], sem),
        pltpu.async_copy(v_cls, o_cls.at[b], sem),
    ]
    for c in cp_out:
        c.wait()


@jax.jit
def kernel(rois, roi_scores, roi_labels, gt_boxes, roi_features):
    rois_q = jnp.transpose(rois, (2, 0, 1))
    gtc_q = jnp.transpose(gt_boxes, (0, 2, 1))
    feat2d = roi_features.reshape(B * R, F)
    perm = jnp.asarray(_PERM)

    f32, i32 = jnp.float32, jnp.int32
    out_type = (
        jax.ShapeDtypeStruct((7, B, ROI_PER_IMAGE), f32),
        jax.ShapeDtypeStruct((7, B, ROI_PER_IMAGE), f32),
        jax.ShapeDtypeStruct((B, ROI_PER_IMAGE), f32),
        jax.ShapeDtypeStruct((B, ROI_PER_IMAGE), f32),
        jax.ShapeDtypeStruct((B, ROI_PER_IMAGE), i32),
        jax.ShapeDtypeStruct((B, ROI_PER_IMAGE, F), f32),
        jax.ShapeDtypeStruct((B, ROI_PER_IMAGE), i32),
        jax.ShapeDtypeStruct((B, ROI_PER_IMAGE), f32),
    )
    scratch = [
        pltpu.VMEM((7, R), f32),
        pltpu.VMEM((8, N), f32),
        pltpu.VMEM((R,), i32),
        pltpu.VMEM((R,), f32),
        pltpu.VMEM((3 * R,), i32),
        pltpu.VMEM((3 * GROWS * 16,), f32),
        pltpu.VMEM((3 * GROWS * 16,), f32),
        pltpu.VMEM((GROWS * 16,), f32),
        pltpu.VMEM((GROWS,), i32),
        pltpu.VMEM((RCLS_PAD,), i32),
        pltpu.VMEM((R,), f32),
        pltpu.VMEM((R,), i32),
        pltpu.VMEM((ROI_PER_IMAGE,), i32),
        pltpu.VMEM((ROI_PER_IMAGE,), i32),
        pltpu.VMEM((7, ROI_PER_IMAGE), f32),
        pltpu.VMEM((7, ROI_PER_IMAGE), f32),
        pltpu.VMEM((ROI_PER_IMAGE,), f32),
        pltpu.VMEM((ROI_PER_IMAGE,), f32),
        pltpu.VMEM((ROI_PER_IMAGE,), i32),
        pltpu.VMEM((ROI_PER_IMAGE,), i32),
        pltpu.VMEM((ROI_PER_IMAGE,), f32),
        pltpu.VMEM((ROI_PER_IMAGE, F), f32),
        pltpu.SemaphoreType.DMA,
    ]
    mesh = plsc.VectorSubcoreMesh(core_axis_name="c", subcore_axis_name="s")
    brq, bgrq, bi, bs, bl, bf, rvm, cls = pl.kernel(
        _body, out_type=out_type, mesh=mesh, scratch_types=scratch,
        compiler_params=pltpu.CompilerParams(needs_layout_passes=False),
    )(rois_q, gtc_q, roi_labels, roi_scores, perm, feat2d)

    br = jnp.transpose(brq, (1, 2, 0))
    bgr = jnp.transpose(bgrq, (1, 2, 0))
    return br, bgr, bi, bs, bl, bf, rvm, cls


# speedup vs baseline: 1.0342x; 1.0342x over previous
import jax
import jax.numpy as jnp
import numpy as np
from jax import lax
from jax.experimental import pallas as pl
from jax.experimental.pallas import tpu as pltpu
from jax.experimental.pallas import tpu_sc as plsc

B, R, N, F = 32, 1024, 64, 128
ROI_PER_IMAGE = 128
FG_NUM = 64
HARD_NUM = 51
EASY_NUM = 13
REG_FG_THRESH = 0.55
CLS_FG_THRESH = 0.75
CLS_BG_THRESH = 0.25
CLS_BG_THRESH_LO = 0.1

NVEC_R = R // 16
NVEC_S = ROI_PER_IMAGE // 16
NVEC_G = N // 16
RCLS_PAD = R + 64
GROWS = 80


def _host_perms() -> np.ndarray:
    u = np.uint32

    def tf(k, n):
        i = np.arange(n, dtype=np.uint64)
        x0 = (i >> np.uint64(32)).astype(u)
        x1 = (i & np.uint64(0xFFFFFFFF)).astype(u)
        rot = ((13, 15, 26, 6), (17, 29, 16, 24))
        ks = (u(k[0]), u(k[1]), u(k[0]) ^ u(k[1]) ^ u(0x1BD11BDA))
        sched = ((ks[1], ks[2]), (ks[2], ks[0]), (ks[0], ks[1]),
                 (ks[1], ks[2]), (ks[2], ks[0]))
        with np.errstate(over="ignore"):
            x0 = x0 + ks[0]
            x1 = x1 + ks[1]
            for gi in range(5):
                for rr in rot[gi % 2]:
                    x0 = x0 + x1
                    x1 = (x1 << u(rr)) | (x1 >> u(32 - rr))
                    x1 = x1 ^ x0
                x0 = x0 + sched[gi][0]
                x1 = x1 + sched[gi][1] + u(gi + 1)
        return x0, x1

    def split(k, n):
        x0, x1 = tf(k, n)
        return list(zip(x0, x1))

    def uniform01(k, n):
        x0, x1 = tf(k, n)
        bits = x0 ^ x1
        return ((bits >> u(9)) | u(0x3F800000)).view(np.float32) - np.float32(1.0)

    r = np.stack([np.stack([uniform01(kj, R) for kj in split(kb, 3)])
                  for kb in split((u(0), u(1)), B)])
    return np.argsort(-r, axis=-1, kind="stable").astype(np.int32)


_PERM = _host_perms().reshape(B, 3 * R)


def _body(rois_hbm, gtc_hbm, lab_hbm, sco_hbm, perm_hbm, feat_hbm,
          o_br, o_bgr, o_bi, o_bs, o_bl, o_bf, o_rvm, o_cls,
          v_rois, v_gtc, v_lab, v_sco, v_perm,
          v_bmin, v_bmax, v_vb, v_gorig, v_rcls,
          v_mo, v_asg, v_idx, v_gidx,
          v_br, v_bgr, v_bi, v_bs, v_bl, v_rvm, v_cls, v_bf,
          sem):
    b = lax.axis_index("c") * 16 + lax.axis_index("s")

    cp_in = [
        pltpu.async_copy(rois_hbm.at[:, b], v_rois, sem),
        pltpu.async_copy(gtc_hbm.at[b], v_gtc, sem),
        pltpu.async_copy(lab_hbm.at[b], v_lab, sem),
        pltpu.async_copy(sco_hbm.at[b], v_sco, sem),
        pltpu.async_copy(perm_hbm.at[b], v_perm, sem),
    ]
    for c in cp_in:
        c.wait()

    lane = lax.iota(jnp.int32, 16)
    zero16i = jnp.full((16,), 0, jnp.int32)
    qv = [zero16i + q for q in range(8)]

    glab = [v_gtc[7, pl.ds(ch * 16, 16)].astype(jnp.int32)
            for ch in range(NVEC_G)]
    gcnt = [jnp.full((16,), 0, jnp.int32) for _ in range(3)]
    for ch in range(NVEC_G):
        for c in range(3):
            gcnt[c] = gcnt[c] + plsc.all_reduce_population_count(glab[ch] == c + 1)
    a0 = (gcnt[0] + 3) & -4
    a1 = (gcnt[1] + 3) & -4
    gstart = [zero16i, a0, a0 + a1]

    grank = [jnp.full((16,), 0, jnp.int32) for _ in range(3)]
    for ch in range(NVEC_G):
        o = ch * 16
        labv = glab[ch]
        dest = zero16i
        for c in range(3):
            m = labv == c + 1
            pc = plsc.cumsum(m.astype(jnp.int32))
            dest = jnp.where(m, gstart[c] + grank[c] + pc - 1, dest)
            grank[c] = grank[c] + plsc.all_reduce_population_count(m)
        plsc.store_scatter(v_gorig, [dest], lane + o)
        c0 = v_gtc[0, pl.ds(o, 16)]
        c1 = v_gtc[1, pl.ds(o, 16)]
        c2 = v_gtc[2, pl.ds(o, 16)]
        s0 = v_gtc[3, pl.ds(o, 16)]
        s1 = v_gtc[4, pl.ds(o, 16)]
        s2 = v_gtc[5, pl.ds(o, 16)]
        rows = (c0 - s0 * 0.5, c1 - s1 * 0.5, c2 - s2 * 0.5,
                c0 + s0 * 0.5, c1 + s1 * 0.5, c2 + s2 * 0.5,
                (s0 * s1) * s2)
        refs = (v_bmin, v_bmin, v_bmin, v_bmax, v_bmax, v_bmax, v_vb)
        offs = (0, GROWS * 16, 2 * GROWS * 16, 0, GROWS * 16, 2 * GROWS * 16, 0)
        dest16 = dest * 16
        for l in range(16):
            idx = dest16 + l
            for rowv, ref, qoff in zip(rows, refs, offs):
                plsc.store_scatter(ref, [idx + qoff], rowv)

    def zero_rcls(v, _):
        v_rcls[pl.ds(v * 16, 16)] = zero16i
        return ()

    lax.fori_loop(0, RCLS_PAD // 16, zero_rcls, ())

    rbase = [zero16i]
    rcnt = []
    for c in range(3):
        base = rbase[c]

        def rpass(v, cnt, c=c, base=base):
            labv = v_lab[pl.ds(v * 16, 16)]
            m = labv == c + 1
            pc = plsc.cumsum(m.astype(jnp.int32))
            slot = base + cnt + pc - 1
            plsc.store_scatter(v_rcls, [slot], lane + v * 16, mask=m)
            return cnt + plsc.all_reduce_population_count(m)

        cnt = lax.fori_loop(0, NVEC_R, rpass, zero16i)
        rcnt.append(cnt)
        rbase.append(base + ((cnt + 15) & -16))

    for c in range(3):
        r_lo = jnp.max(rbase[c])
        r_n = jnp.max(rcnt[c])
        g_lo = jnp.max(gstart[c])
        g_hi = g_lo + jnp.max(gcnt[c])
        g_n4 = (g_hi - g_lo + 3) // 4
        nchunk = (r_n + 15) // 16

        def chunk_body(j, _, r_lo=r_lo, r_n=r_n, g_lo=g_lo, g_hi=g_hi,
                       g_n4=g_n4):
            idxv = v_rcls[pl.ds(r_lo + j * 16, 16)]
            lanemask = lane < (r_n - j * 16)
            cx = plsc.load_gather(v_rois, [qv[0], idxv])
            cy = plsc.load_gather(v_rois, [qv[1], idxv])
            cz = plsc.load_gather(v_rois, [qv[2], idxv])
            dx = plsc.load_gather(v_rois, [qv[3], idxv])
            dy = plsc.load_gather(v_rois, [qv[4], idxv])
            dz = plsc.load_gather(v_rois, [qv[5], idxv])
            hx = dx * 0.5
            hy = dy * 0.5
            hz = dz * 0.5
            ax0 = cx - hx
            ax1 = cx + hx
            ay0 = cy - hy
            ay1 = cy + hy
            az0 = cz - hz
            az1 = cz + hz
            va = dx * dy * dz

            def inner(g, carry):
                bn, bd, bidx = carry
                go = g * 16
                ix = jnp.maximum(jnp.minimum(ax1, v_bmax[pl.ds(go, 16)])
                                 - jnp.maximum(ax0, v_bmin[pl.ds(go, 16)]), 0.0)
                iy = jnp.maximum(jnp.minimum(ay1, v_bmax[pl.ds(GROWS * 16 + go, 16)])
                                 - jnp.maximum(ay0, v_bmin[pl.ds(GROWS * 16 + go, 16)]), 0.0)
                iz = jnp.maximum(jnp.minimum(az1, v_bmax[pl.ds(2 * GROWS * 16 + go, 16)])
                                 - jnp.maximum(az0, v_bmin[pl.ds(2 * GROWS * 16 + go, 16)]), 0.0)
                iv = (ix * iy) * iz
                den = jnp.maximum(va + v_vb[pl.ds(go, 16)] - iv, 1e-6)
                better = iv * bd > bn * den
                gvec = zero16i + g
                bn = jnp.where(better, iv, bn)
                bd = jnp.where(better, den, bd)
                bidx = jnp.where(better, gvec, bidx)
                return bn, bd, bidx

            init = (jnp.full((16,), 0.0, jnp.float32),
                    jnp.full((16,), 1.0, jnp.float32),
                    jnp.full((16,), -1, jnp.int32))
            bn, bd, bidx = lax.fori_loop(g_lo, g_hi, inner, init)
            mo = bn / bd
            asg = plsc.load_gather(v_gorig, [jnp.maximum(bidx, 0)])
            asg = jnp.where(bidx < 0, 0, asg)
            plsc.store_scatter(v_mo, [idxv], mo, mask=lanemask)
            plsc.store_scatter(v_asg, [idxv], asg, mask=lanemask)
            return ()

        lax.fori_loop(0, nchunk, chunk_body, ())

    def sample_pool(pool, off, k):
        thresholds = {
            0: lambda v: v >= REG_FG_THRESH,
            1: lambda v: jnp.logical_and(v < REG_FG_THRESH, v >= CLS_BG_THRESH_LO),
            2: lambda v: v < CLS_BG_THRESH_LO,
        }
        mask_fn = thresholds[pool]

        def pass_a(v, cnt):
            pidx = v_perm[pl.ds(pool * R + v * 16, 16)]
            vals = plsc.load_gather(v_mo, [pidx])
            m = mask_fn(vals)
            pc = plsc.cumsum(m.astype(jnp.int32))
            slot = cnt + pc - 1
            ok = jnp.logical_and(m, slot < k)
            plsc.store_scatter(v_idx, [slot + off], pidx, mask=ok)
            return cnt + plsc.all_reduce_population_count(m)

        cnt = lax.fori_loop(0, NVEC_R, pass_a, zero16i)

        def pass_b(v, cnt):
            pidx = lane + v * 16
            vals = v_mo[pl.ds(v * 16, 16)]
            m = jnp.logical_not(mask_fn(vals))
            pc = plsc.cumsum(m.astype(jnp.int32))
            slot = cnt + pc - 1
            ok = jnp.logical_and(m, slot < k)
            plsc.store_scatter(v_idx, [slot + off], pidx, mask=ok)
            return cnt + plsc.all_reduce_population_count(m)

        @pl.when(jnp.max(cnt) < k)
        def _():
            lax.fori_loop(0, NVEC_R, pass_b, cnt)

    sample_pool(0, 0, FG_NUM)
    sample_pool(1, FG_NUM, HARD_NUM)
    sample_pool(2, FG_NUM + HARD_NUM, EASY_NUM)

    for s in range(NVEC_S):
        sl = pl.ds(s * 16, 16)
        sidx = v_idx[sl]
        iou_s = plsc.load_gather(v_mo, [sidx])
        v_bi[sl] = iou_s
        v_rvm[sl] = (iou_s > REG_FG_THRESH).astype(jnp.int32)
        fgm = iou_s > CLS_FG_THRESH
        bgm = iou_s < CLS_BG_THRESH
        interval = jnp.logical_and(jnp.logical_not(fgm), jnp.logical_not(bgm))
        v_cls[sl] = jnp.where(interval, (iou_s - CLS_BG_THRESH) * 2.0,
                              jnp.where(fgm, 1.0, 0.0))
        v_bl[sl] = plsc.load_gather(v_lab, [sidx])
        v_bs[sl] = plsc.load_gather(v_sco, [sidx])
        asgs = plsc.load_gather(v_asg, [sidx])
        for dd in range(7):
            v_br[dd, sl] = plsc.load_gather(v_rois, [qv[dd], sidx])
            v_bgr[dd, sl] = plsc.load_gather(v_gtc, [qv[dd], asgs])
        v_gidx[sl] = sidx + b * R

    pltpu.async_copy(feat_hbm.at[v_gidx], v_bf, sem).wait()

    cp_out = [
        pltpu.async_copy(v_br, o_br.at[:, b], sem),
        pltpu.async_copy(v_bgr, o_bgr.at[:, b], sem),
        pltpu.async_copy(v_bi, o_bi.at[b], sem),
        pltpu.async_copy(v_bs, o_bs.at[b], sem),
        pltpu.async_copy(v_bl, o_bl.at[b], sem),
        pltpu.async_copy(v_bf, o_bf.at[b], sem),
        pltpu.async_copy(v_rvm, o_rvm.at[b], sem),
        pltpu.async_copy(v_cls, o_cls.at[b], sem),
    ]
    for c in cp_out:
        c.wait()


@jax.jit
def kernel(rois, roi_scores, roi_labels, gt_boxes, roi_features):
    rois_q = jnp.transpose(rois, (2, 0, 1))
    gtc_q = jnp.transpose(gt_boxes, (0, 2, 1))
    feat2d = roi_features.reshape(B * R, F)
    perm = jnp.asarray(_PERM)

    f32, i32 = jnp.float32, jnp.int32
    out_type = (
        jax.ShapeDtypeStruct((7, B, ROI_PER_IMAGE), f32),
        jax.ShapeDtypeStruct((7, B, ROI_PER_IMAGE), f32),
        jax.ShapeDtypeStruct((B, ROI_PER_IMAGE), f32),
        jax.ShapeDtypeStruct((B, ROI_PER_IMAGE), f32),
        jax.ShapeDtypeStruct((B, ROI_PER_IMAGE), i32),
        jax.ShapeDtypeStruct((B, ROI_PER_IMAGE, F), f32),
        jax.ShapeDtypeStruct((B, ROI_PER_IMAGE), i32),
        jax.ShapeDtypeStruct((B, ROI_PER_IMAGE), f32),
    )
    scratch = [
        pltpu.VMEM((7, R), f32),
        pltpu.VMEM((8, N), f32),
        pltpu.VMEM((R,), i32),
        pltpu.VMEM((R,), f32),
        pltpu.VMEM((3 * R,), i32),
        pltpu.VMEM((3 * GROWS * 16,), f32),
        pltpu.VMEM((3 * GROWS * 16,), f32),
        pltpu.VMEM((GROWS * 16,), f32),
        pltpu.VMEM((GROWS,), i32),
        pltpu.VMEM((RCLS_PAD,), i32),
        pltpu.VMEM((R,), f32),
        pltpu.VMEM((R,), i32),
        pltpu.VMEM((ROI_PER_IMAGE,), i32),
        pltpu.VMEM((ROI_PER_IMAGE,), i32),
        pltpu.VMEM((7, ROI_PER_IMAGE), f32),
        pltpu.VMEM((7, ROI_PER_IMAGE), f32),
        pltpu.VMEM((ROI_PER_IMAGE,), f32),
        pltpu.VMEM((ROI_PER_IMAGE,), f32),
        pltpu.VMEM((ROI_PER_IMAGE,), i32),
        pltpu.VMEM((ROI_PER_IMAGE,), i32),
        pltpu.VMEM((ROI_PER_IMAGE,), f32),
        pltpu.VMEM((ROI_PER_IMAGE, F), f32),
        pltpu.SemaphoreType.DMA,
    ]
    mesh = plsc.VectorSubcoreMesh(core_axis_name="c", subcore_axis_name="s")
    brq, bgrq, bi, bs, bl, bf, rvm, cls = pl.kernel(
        _body, out_type=out_type, mesh=mesh, scratch_types=scratch,
        compiler_params=pltpu.CompilerParams(needs_layout_passes=False),
    )(rois_q, gtc_q, roi_labels, roi_scores, perm, feat2d)

    br = jnp.transpose(brq, (1, 2, 0))
    bgr = jnp.transpose(bgrq, (1, 2, 0))
    return br, bgr, bi, bs, bl, bf, rvm, cls


# speedup vs baseline: 1.0754x; 1.0398x over previous
import jax
import jax.numpy as jnp
import numpy as np
from jax import lax
from jax.experimental import pallas as pl
from jax.experimental.pallas import tpu as pltpu
from jax.experimental.pallas import tpu_sc as plsc

B, R, N, F = 32, 1024, 64, 128
ROI_PER_IMAGE = 128
FG_NUM = 64
HARD_NUM = 51
EASY_NUM = 13
REG_FG_THRESH = 0.55
CLS_FG_THRESH = 0.75
CLS_BG_THRESH = 0.25
CLS_BG_THRESH_LO = 0.1

NVEC_R = R // 16
NVEC_S = ROI_PER_IMAGE // 16
NVEC_G = N // 16
GROWS = 80


def _host_perms() -> np.ndarray:
    u = np.uint32

    def tf(k, n):
        i = np.arange(n, dtype=np.uint64)
        x0 = (i >> np.uint64(32)).astype(u)
        x1 = (i & np.uint64(0xFFFFFFFF)).astype(u)
        rot = ((13, 15, 26, 6), (17, 29, 16, 24))
        ks = (u(k[0]), u(k[1]), u(k[0]) ^ u(k[1]) ^ u(0x1BD11BDA))
        sched = ((ks[1], ks[2]), (ks[2], ks[0]), (ks[0], ks[1]),
                 (ks[1], ks[2]), (ks[2], ks[0]))
        with np.errstate(over="ignore"):
            x0 = x0 + ks[0]
            x1 = x1 + ks[1]
            for gi in range(5):
                for rr in rot[gi % 2]:
                    x0 = x0 + x1
                    x1 = (x1 << u(rr)) | (x1 >> u(32 - rr))
                    x1 = x1 ^ x0
                x0 = x0 + sched[gi][0]
                x1 = x1 + sched[gi][1] + u(gi + 1)
        return x0, x1

    def split(k, n):
        x0, x1 = tf(k, n)
        return list(zip(x0, x1))

    def uniform01(k, n):
        x0, x1 = tf(k, n)
        bits = x0 ^ x1
        return ((bits >> u(9)) | u(0x3F800000)).view(np.float32) - np.float32(1.0)

    r = np.stack([np.stack([uniform01(kj, R) for kj in split(kb, 3)])
                  for kb in split((u(0), u(1)), B)])
    return np.argsort(-r, axis=-1, kind="stable").astype(np.int32)


_PERM = _host_perms().reshape(B, 3 * R)


def _body(rois_hbm, gtc_hbm, lab_hbm, sco_hbm, perm_hbm, feat_hbm,
          o_br, o_bgr, o_bi, o_bs, o_bl, o_bf, o_rvm, o_cls,
          v_rois, v_gtc, v_lab, v_sco, v_perm,
          v_bmin, v_bmax, v_vb, v_gorig, v_rcls,
          v_mo, v_asg, v_idx, v_gidx,
          v_br, v_bgr, v_bi, v_bs, v_bl, v_rvm, v_cls, v_bf,
          sem):
    b = lax.axis_index("c") * 16 + lax.axis_index("s")

    cp_in = [
        pltpu.async_copy(rois_hbm.at[:, b], v_rois, sem),
        pltpu.async_copy(gtc_hbm.at[b], v_gtc, sem),
        pltpu.async_copy(lab_hbm.at[b], v_lab, sem),
        pltpu.async_copy(sco_hbm.at[b], v_sco, sem),
        pltpu.async_copy(perm_hbm.at[b], v_perm, sem),
    ]
    for c in cp_in:
        c.wait()

    lane = lax.iota(jnp.int32, 16)
    zero16i = jnp.full((16,), 0, jnp.int32)
    qv = [zero16i + q for q in range(8)]

    glab = [v_gtc[7, pl.ds(ch * 16, 16)].astype(jnp.int32)
            for ch in range(NVEC_G)]
    gcnt = [jnp.full((16,), 0, jnp.int32) for _ in range(3)]
    for ch in range(NVEC_G):
        for c in range(3):
            gcnt[c] = gcnt[c] + plsc.all_reduce_population_count(glab[ch] == c + 1)
    a0 = (gcnt[0] + 3) & -4
    a1 = (gcnt[1] + 3) & -4
    gstart = [zero16i, a0, a0 + a1]

    grank = [jnp.full((16,), 0, jnp.int32) for _ in range(3)]
    for ch in range(NVEC_G):
        o = ch * 16
        labv = glab[ch]
        dest = zero16i
        for c in range(3):
            m = labv == c + 1
            pc = plsc.cumsum(m.astype(jnp.int32))
            dest = jnp.where(m, gstart[c] + grank[c] + pc - 1, dest)
            grank[c] = grank[c] + plsc.all_reduce_population_count(m)
        plsc.store_scatter(v_gorig, [dest], lane + o)
        c0 = v_gtc[0, pl.ds(o, 16)]
        c1 = v_gtc[1, pl.ds(o, 16)]
        c2 = v_gtc[2, pl.ds(o, 16)]
        s0 = v_gtc[3, pl.ds(o, 16)]
        s1 = v_gtc[4, pl.ds(o, 16)]
        s2 = v_gtc[5, pl.ds(o, 16)]
        rows = (c0 - s0 * 0.5, c1 - s1 * 0.5, c2 - s2 * 0.5,
                c0 + s0 * 0.5, c1 + s1 * 0.5, c2 + s2 * 0.5,
                (s0 * s1) * s2)
        refs = (v_bmin, v_bmin, v_bmin, v_bmax, v_bmax, v_bmax, v_vb)
        offs = (0, GROWS * 16, 2 * GROWS * 16, 0, GROWS * 16, 2 * GROWS * 16, 0)
        dest16 = dest * 16
        for l in range(16):
            idx = dest16 + l
            for rowv, ref, qoff in zip(rows, refs, offs):
                plsc.store_scatter(ref, [idx + qoff], rowv)

    def rpass(v, carry):
        labv = v_lab[pl.ds(v * 16, 16)]
        idxvec = lane + v * 16
        out = []
        for c in range(3):
            m = labv == c + 1
            pc = plsc.cumsum(m.astype(jnp.int32))
            slot = carry[c] + pc - 1
            plsc.store_scatter(v_rcls, [slot + c * R], idxvec, mask=m)
            out.append(carry[c] + plsc.all_reduce_population_count(m))
        return tuple(out)

    rcnt = lax.fori_loop(0, NVEC_R, rpass, (zero16i, zero16i, zero16i))

    for c in range(3):
        r_lo = c * R
        r_n = jnp.max(rcnt[c])
        g_lo = jnp.max(gstart[c])
        g_hi = g_lo + jnp.max(gcnt[c])
        nchunk = (r_n + 15) // 16

        def chunk_body(j, _, r_lo=r_lo, r_n=r_n, g_lo=g_lo, g_hi=g_hi):
            idxv = v_rcls[pl.ds(r_lo + j * 16, 16)] & (R - 1)
            lanemask = lane < (r_n - j * 16)
            cx = plsc.load_gather(v_rois, [qv[0], idxv])
            cy = plsc.load_gather(v_rois, [qv[1], idxv])
            cz = plsc.load_gather(v_rois, [qv[2], idxv])
            dx = plsc.load_gather(v_rois, [qv[3], idxv])
            dy = plsc.load_gather(v_rois, [qv[4], idxv])
            dz = plsc.load_gather(v_rois, [qv[5], idxv])
            hx = dx * 0.5
            hy = dy * 0.5
            hz = dz * 0.5
            ax0 = cx - hx
            ax1 = cx + hx
            ay0 = cy - hy
            ay1 = cy + hy
            az0 = cz - hz
            az1 = cz + hz
            va = dx * dy * dz

            def inner(g, carry):
                bn, bd, bidx = carry
                go = g * 16
                ix = jnp.maximum(jnp.minimum(ax1, v_bmax[pl.ds(go, 16)])
                                 - jnp.maximum(ax0, v_bmin[pl.ds(go, 16)]), 0.0)
                iy = jnp.maximum(jnp.minimum(ay1, v_bmax[pl.ds(GROWS * 16 + go, 16)])
                                 - jnp.maximum(ay0, v_bmin[pl.ds(GROWS * 16 + go, 16)]), 0.0)
                iz = jnp.maximum(jnp.minimum(az1, v_bmax[pl.ds(2 * GROWS * 16 + go, 16)])
                                 - jnp.maximum(az0, v_bmin[pl.ds(2 * GROWS * 16 + go, 16)]), 0.0)
                iv = (ix * iy) * iz
                den = jnp.maximum(va + v_vb[pl.ds(go, 16)] - iv, 1e-6)
                better = iv * bd > bn * den
                gvec = zero16i + g
                bn = jnp.where(better, iv, bn)
                bd = jnp.where(better, den, bd)
                bidx = jnp.where(better, gvec, bidx)
                return bn, bd, bidx

            init = (jnp.full((16,), 0.0, jnp.float32),
                    jnp.full((16,), 1.0, jnp.float32),
                    jnp.full((16,), -1, jnp.int32))
            bn, bd, bidx = lax.fori_loop(g_lo, g_hi, inner, init)
            mo = bn / bd
            asg = plsc.load_gather(v_gorig, [jnp.maximum(bidx, 0)])
            asg = jnp.where(bidx < 0, 0, asg)
            plsc.store_scatter(v_mo, [idxv], mo, mask=lanemask)
            plsc.store_scatter(v_asg, [idxv], asg, mask=lanemask)
            return ()

        lax.fori_loop(0, nchunk, chunk_body, ())

    def sample_pool(pool, off, k):
        thresholds = {
            0: lambda v: v >= REG_FG_THRESH,
            1: lambda v: jnp.logical_and(v < REG_FG_THRESH, v >= CLS_BG_THRESH_LO),
            2: lambda v: v < CLS_BG_THRESH_LO,
        }
        mask_fn = thresholds[pool]

        def pass_a(v, cnt):
            pidx = v_perm[pl.ds(pool * R + v * 16, 16)]
            vals = plsc.load_gather(v_mo, [pidx])
            m = mask_fn(vals)
            pc = plsc.cumsum(m.astype(jnp.int32))
            slot = cnt + pc - 1
            ok = jnp.logical_and(m, slot < k)
            plsc.store_scatter(v_idx, [slot + off], pidx, mask=ok)
            return cnt + plsc.all_reduce_population_count(m)

        cnt = lax.fori_loop(0, NVEC_R, pass_a, zero16i)

        def pass_b(v, cnt):
            pidx = lane + v * 16
            vals = v_mo[pl.ds(v * 16, 16)]
            m = jnp.logical_not(mask_fn(vals))
            pc = plsc.cumsum(m.astype(jnp.int32))
            slot = cnt + pc - 1
            ok = jnp.logical_and(m, slot < k)
            plsc.store_scatter(v_idx, [slot + off], pidx, mask=ok)
            return cnt + plsc.all_reduce_population_count(m)

        @pl.when(jnp.max(cnt) < k)
        def _():
            lax.fori_loop(0, NVEC_R, pass_b, cnt)

    sample_pool(0, 0, FG_NUM)
    sample_pool(1, FG_NUM, HARD_NUM)
    sample_pool(2, FG_NUM + HARD_NUM, EASY_NUM)

    for s in range(NVEC_S):
        sl = pl.ds(s * 16, 16)
        sidx = v_idx[sl]
        iou_s = plsc.load_gather(v_mo, [sidx])
        v_bi[sl] = iou_s
        v_rvm[sl] = (iou_s > REG_FG_THRESH).astype(jnp.int32)
        fgm = iou_s > CLS_FG_THRESH
        bgm = iou_s < CLS_BG_THRESH
        interval = jnp.logical_and(jnp.logical_not(fgm), jnp.logical_not(bgm))
        v_cls[sl] = jnp.where(interval, (iou_s - CLS_BG_THRESH) * 2.0,
                              jnp.where(fgm, 1.0, 0.0))
        v_bl[sl] = plsc.load_gather(v_lab, [sidx])
        v_bs[sl] = plsc.load_gather(v_sco, [sidx])
        asgs = plsc.load_gather(v_asg, [sidx])
        for dd in range(7):
            v_br[dd, sl] = plsc.load_gather(v_rois, [qv[dd], sidx])
            v_bgr[dd, sl] = plsc.load_gather(v_gtc, [qv[dd], asgs])
        v_gidx[sl] = sidx + b * R

    pltpu.async_copy(feat_hbm.at[v_gidx], v_bf, sem).wait()

    cp_out = [
        pltpu.async_copy(v_br, o_br.at[:, b], sem),
        pltpu.async_copy(v_bgr, o_bgr.at[:, b], sem),
        pltpu.async_copy(v_bi, o_bi.at[b], sem),
        pltpu.async_copy(v_bs, o_bs.at[b], sem),
        pltpu.async_copy(v_bl, o_bl.at[b], sem),
        pltpu.async_copy(v_bf, o_bf.at[b], sem),
        pltpu.async_copy(v_rvm, o_rvm.at[b], sem),
        pltpu.async_copy(v_cls, o_cls.at[b], sem),
    ]
    for c in cp_out:
        c.wait()


@jax.jit
def kernel(rois, roi_scores, roi_labels, gt_boxes, roi_features):
    rois_q = jnp.transpose(rois, (2, 0, 1))
    gtc_q = jnp.transpose(gt_boxes, (0, 2, 1))
    feat2d = roi_features.reshape(B * R, F)
    perm = jnp.asarray(_PERM)

    f32, i32 = jnp.float32, jnp.int32
    out_type = (
        jax.ShapeDtypeStruct((7, B, ROI_PER_IMAGE), f32),
        jax.ShapeDtypeStruct((7, B, ROI_PER_IMAGE), f32),
        jax.ShapeDtypeStruct((B, ROI_PER_IMAGE), f32),
        jax.ShapeDtypeStruct((B, ROI_PER_IMAGE), f32),
        jax.ShapeDtypeStruct((B, ROI_PER_IMAGE), i32),
        jax.ShapeDtypeStruct((B, ROI_PER_IMAGE, F), f32),
        jax.ShapeDtypeStruct((B, ROI_PER_IMAGE), i32),
        jax.ShapeDtypeStruct((B, ROI_PER_IMAGE), f32),
    )
    scratch = [
        pltpu.VMEM((7, R), f32),
        pltpu.VMEM((8, N), f32),
        pltpu.VMEM((R,), i32),
        pltpu.VMEM((R,), f32),
        pltpu.VMEM((3 * R,), i32),
        pltpu.VMEM((3 * GROWS * 16,), f32),
        pltpu.VMEM((3 * GROWS * 16,), f32),
        pltpu.VMEM((GROWS * 16,), f32),
        pltpu.VMEM((GROWS,), i32),
        pltpu.VMEM((3 * R,), i32),
        pltpu.VMEM((R,), f32),
        pltpu.VMEM((R,), i32),
        pltpu.VMEM((ROI_PER_IMAGE,), i32),
        pltpu.VMEM((ROI_PER_IMAGE,), i32),
        pltpu.VMEM((7, ROI_PER_IMAGE), f32),
        pltpu.VMEM((7, ROI_PER_IMAGE), f32),
        pltpu.VMEM((ROI_PER_IMAGE,), f32),
        pltpu.VMEM((ROI_PER_IMAGE,), f32),
        pltpu.VMEM((ROI_PER_IMAGE,), i32),
        pltpu.VMEM((ROI_PER_IMAGE,), i32),
        pltpu.VMEM((ROI_PER_IMAGE,), f32),
        pltpu.VMEM((ROI_PER_IMAGE, F), f32),
        pltpu.SemaphoreType.DMA,
    ]
    mesh = plsc.VectorSubcoreMesh(core_axis_name="c", subcore_axis_name="s")
    brq, bgrq, bi, bs, bl, bf, rvm, cls = pl.kernel(
        _body, out_type=out_type, mesh=mesh, scratch_types=scratch,
        compiler_params=pltpu.CompilerParams(needs_layout_passes=False),
    )(rois_q, gtc_q, roi_labels, roi_scores, perm, feat2d)

    br = jnp.transpose(brq, (1, 2, 0))
    bgr = jnp.transpose(bgrq, (1, 2, 0))
    return br, bgr, bi, bs, bl, bf, rvm, cls


# speedup vs baseline: 1.0790x; 1.0034x over previous
import jax
import jax.numpy as jnp
import numpy as np
from jax import lax
from jax.experimental import pallas as pl
from jax.experimental.pallas import tpu as pltpu
from jax.experimental.pallas import tpu_sc as plsc

B, R, N, F = 32, 1024, 64, 128
ROI_PER_IMAGE = 128
FG_NUM = 64
HARD_NUM = 51
EASY_NUM = 13
REG_FG_THRESH = 0.55
CLS_FG_THRESH = 0.75
CLS_BG_THRESH = 0.25
CLS_BG_THRESH_LO = 0.1

NVEC_R = R // 16
NVEC_S = ROI_PER_IMAGE // 16
NVEC_G = N // 16
GROWS = 80


def _host_perms() -> np.ndarray:
    u = np.uint32

    def tf(k, n):
        i = np.arange(n, dtype=np.uint64)
        x0 = (i >> np.uint64(32)).astype(u)
        x1 = (i & np.uint64(0xFFFFFFFF)).astype(u)
        rot = ((13, 15, 26, 6), (17, 29, 16, 24))
        ks = (u(k[0]), u(k[1]), u(k[0]) ^ u(k[1]) ^ u(0x1BD11BDA))
        sched = ((ks[1], ks[2]), (ks[2], ks[0]), (ks[0], ks[1]),
                 (ks[1], ks[2]), (ks[2], ks[0]))
        with np.errstate(over="ignore"):
            x0 = x0 + ks[0]
            x1 = x1 + ks[1]
            for gi in range(5):
                for rr in rot[gi % 2]:
                    x0 = x0 + x1
                    x1 = (x1 << u(rr)) | (x1 >> u(32 - rr))
                    x1 = x1 ^ x0
                x0 = x0 + sched[gi][0]
                x1 = x1 + sched[gi][1] + u(gi + 1)
        return x0, x1

    def split(k, n):
        x0, x1 = tf(k, n)
        return list(zip(x0, x1))

    def uniform01(k, n):
        x0, x1 = tf(k, n)
        bits = x0 ^ x1
        return ((bits >> u(9)) | u(0x3F800000)).view(np.float32) - np.float32(1.0)

    r = np.stack([np.stack([uniform01(kj, R) for kj in split(kb, 3)])
                  for kb in split((u(0), u(1)), B)])
    return np.argsort(-r, axis=-1, kind="stable").astype(np.int32)


_PERM = _host_perms().reshape(B, 3 * R)


def _body(rois_hbm, gtc_hbm, lab_hbm, sco_hbm, perm_hbm, feat_hbm,
          o_br, o_bgr, o_bi, o_bs, o_bl, o_bf, o_rvm, o_cls,
          v_rois, v_gtc, v_lab, v_sco, v_perm,
          v_bmin, v_bmax, v_vb, v_gorig, v_rcls,
          v_mo, v_asg, v_idx, v_gidx,
          v_br, v_bgr, v_bi, v_bs, v_bl, v_rvm, v_cls, v_bf,
          sem):
    b = lax.axis_index("c") * 16 + lax.axis_index("s")

    cp_in = [
        pltpu.async_copy(rois_hbm.at[:, b], v_rois, sem),
        pltpu.async_copy(gtc_hbm.at[b], v_gtc, sem),
        pltpu.async_copy(lab_hbm.at[b], v_lab, sem),
        pltpu.async_copy(sco_hbm.at[b], v_sco, sem),
        pltpu.async_copy(perm_hbm.at[b], v_perm, sem),
    ]
    for c in cp_in:
        c.wait()

    lane = lax.iota(jnp.int32, 16)
    zero16i = jnp.full((16,), 0, jnp.int32)
    qv = [zero16i + q for q in range(8)]

    glab = [v_gtc[7, pl.ds(ch * 16, 16)].astype(jnp.int32)
            for ch in range(NVEC_G)]
    gcnt = [jnp.full((16,), 0, jnp.int32) for _ in range(3)]
    for ch in range(NVEC_G):
        for c in range(3):
            gcnt[c] = gcnt[c] + plsc.all_reduce_population_count(glab[ch] == c + 1)
    a0 = (gcnt[0] + 3) & -4
    a1 = (gcnt[1] + 3) & -4
    gstart = [zero16i, a0, a0 + a1]

    grank = [jnp.full((16,), 0, jnp.int32) for _ in range(3)]
    for ch in range(NVEC_G):
        o = ch * 16
        labv = glab[ch]
        dest = zero16i
        for c in range(3):
            m = labv == c + 1
            pc = plsc.cumsum(m.astype(jnp.int32))
            dest = jnp.where(m, gstart[c] + grank[c] + pc - 1, dest)
            grank[c] = grank[c] + plsc.all_reduce_population_count(m)
        plsc.store_scatter(v_gorig, [dest], lane + o)
        c0 = v_gtc[0, pl.ds(o, 16)]
        c1 = v_gtc[1, pl.ds(o, 16)]
        c2 = v_gtc[2, pl.ds(o, 16)]
        s0 = v_gtc[3, pl.ds(o, 16)]
        s1 = v_gtc[4, pl.ds(o, 16)]
        s2 = v_gtc[5, pl.ds(o, 16)]
        rows = (c0 - s0 * 0.5, c1 - s1 * 0.5, c2 - s2 * 0.5,
                c0 + s0 * 0.5, c1 + s1 * 0.5, c2 + s2 * 0.5,
                (s0 * s1) * s2)
        refs = (v_bmin, v_bmin, v_bmin, v_bmax, v_bmax, v_bmax, v_vb)
        offs = (0, GROWS * 16, 2 * GROWS * 16, 0, GROWS * 16, 2 * GROWS * 16, 0)
        dest16 = dest * 16
        for l in range(16):
            idx = dest16 + l
            for rowv, ref, qoff in zip(rows, refs, offs):
                plsc.store_scatter(ref, [idx + qoff], rowv)

    def rpass(v, carry):
        labv = v_lab[pl.ds(v * 16, 16)]
        idxvec = lane + v * 16
        out = []
        for c in range(3):
            m = labv == c + 1
            pc = plsc.cumsum(m.astype(jnp.int32))
            slot = carry[c] + pc - 1
            plsc.store_scatter(v_rcls, [slot + c * R], idxvec, mask=m)
            out.append(carry[c] + plsc.all_reduce_population_count(m))
        return tuple(out)

    rcnt = lax.fori_loop(0, NVEC_R, rpass, (zero16i, zero16i, zero16i))

    for c in range(3):
        r_lo = c * R
        r_n = jnp.max(rcnt[c])
        g_lo = jnp.max(gstart[c])
        g_hi = g_lo + jnp.max(gcnt[c])
        nchunk = (r_n + 15) // 16

        def chunk_body(j, _, r_lo=r_lo, r_n=r_n, g_lo=g_lo, g_hi=g_hi):
            idxv = v_rcls[pl.ds(r_lo + j * 16, 16)] & (R - 1)
            lanemask = lane < (r_n - j * 16)
            cx = plsc.load_gather(v_rois, [qv[0], idxv])
            cy = plsc.load_gather(v_rois, [qv[1], idxv])
            cz = plsc.load_gather(v_rois, [qv[2], idxv])
            dx = plsc.load_gather(v_rois, [qv[3], idxv])
            dy = plsc.load_gather(v_rois, [qv[4], idxv])
            dz = plsc.load_gather(v_rois, [qv[5], idxv])
            hx = dx * 0.5
            hy = dy * 0.5
            hz = dz * 0.5
            ax0 = cx - hx
            ax1 = cx + hx
            ay0 = cy - hy
            ay1 = cy + hy
            az0 = cz - hz
            az1 = cz + hz
            va = dx * dy * dz

            def inner(g, carry):
                bn, bd, bidx = carry
                go = g * 16
                ix = jnp.maximum(jnp.minimum(ax1, v_bmax[pl.ds(go, 16)])
                                 - jnp.maximum(ax0, v_bmin[pl.ds(go, 16)]), 0.0)
                iy = jnp.maximum(jnp.minimum(ay1, v_bmax[pl.ds(GROWS * 16 + go, 16)])
                                 - jnp.maximum(ay0, v_bmin[pl.ds(GROWS * 16 + go, 16)]), 0.0)
                iz = jnp.maximum(jnp.minimum(az1, v_bmax[pl.ds(2 * GROWS * 16 + go, 16)])
                                 - jnp.maximum(az0, v_bmin[pl.ds(2 * GROWS * 16 + go, 16)]), 0.0)
                iv = (ix * iy) * iz
                den = jnp.maximum(va + v_vb[pl.ds(go, 16)] - iv, 1e-6)
                better = iv * bd > bn * den
                gvec = zero16i + g
                bn = jnp.where(better, iv, bn)
                bd = jnp.where(better, den, bd)
                bidx = jnp.where(better, gvec, bidx)
                return bn, bd, bidx

            init = (jnp.full((16,), 0.0, jnp.float32),
                    jnp.full((16,), 1.0, jnp.float32),
                    jnp.full((16,), -1, jnp.int32))
            bn, bd, bidx = lax.fori_loop(g_lo, g_hi, inner, init)
            mo = bn / bd
            asg = plsc.load_gather(v_gorig, [jnp.maximum(bidx, 0)])
            asg = jnp.where(bidx < 0, 0, asg)
            plsc.store_scatter(v_mo, [idxv], mo, mask=lanemask)
            plsc.store_scatter(v_asg, [idxv], asg, mask=lanemask)
            return ()

        lax.fori_loop(0, nchunk, chunk_body, ())

    thresholds = [
        lambda v: v >= REG_FG_THRESH,
        lambda v: jnp.logical_and(v < REG_FG_THRESH, v >= CLS_BG_THRESH_LO),
        lambda v: v < CLS_BG_THRESH_LO,
    ]
    pool_off = (0, FG_NUM, FG_NUM + HARD_NUM)
    pool_k = (FG_NUM, HARD_NUM, EASY_NUM)

    def pass_a(v, carry):
        out = []
        for pool in range(3):
            pidx = v_perm[pl.ds(pool * R + v * 16, 16)]
            vals = plsc.load_gather(v_mo, [pidx])
            m = thresholds[pool](vals)
            pc = plsc.cumsum(m.astype(jnp.int32))
            slot = carry[pool] + pc - 1
            ok = jnp.logical_and(m, slot < pool_k[pool])
            plsc.store_scatter(v_idx, [slot + pool_off[pool]], pidx, mask=ok)
            out.append(carry[pool] + plsc.all_reduce_population_count(m))
        return tuple(out)

    cnt3 = lax.fori_loop(0, NVEC_R, pass_a, (zero16i, zero16i, zero16i))

    for pool in range(3):
        off, k, cnt = pool_off[pool], pool_k[pool], cnt3[pool]
        mask_fn = thresholds[pool]

        def pass_b(v, cnt, mask_fn=mask_fn, off=off, k=k):
            pidx = lane + v * 16
            vals = v_mo[pl.ds(v * 16, 16)]
            m = jnp.logical_not(mask_fn(vals))
            pc = plsc.cumsum(m.astype(jnp.int32))
            slot = cnt + pc - 1
            ok = jnp.logical_and(m, slot < k)
            plsc.store_scatter(v_idx, [slot + off], pidx, mask=ok)
            return cnt + plsc.all_reduce_population_count(m)

        @pl.when(jnp.max(cnt) < k)
        def _(cnt=cnt):
            lax.fori_loop(0, NVEC_R, pass_b, cnt)

    for s in range(NVEC_S):
        sl = pl.ds(s * 16, 16)
        sidx = v_idx[sl]
        iou_s = plsc.load_gather(v_mo, [sidx])
        v_bi[sl] = iou_s
        v_rvm[sl] = (iou_s > REG_FG_THRESH).astype(jnp.int32)
        fgm = iou_s > CLS_FG_THRESH
        bgm = iou_s < CLS_BG_THRESH
        interval = jnp.logical_and(jnp.logical_not(fgm), jnp.logical_not(bgm))
        v_cls[sl] = jnp.where(interval, (iou_s - CLS_BG_THRESH) * 2.0,
                              jnp.where(fgm, 1.0, 0.0))
        v_bl[sl] = plsc.load_gather(v_lab, [sidx])
        v_bs[sl] = plsc.load_gather(v_sco, [sidx])
        asgs = plsc.load_gather(v_asg, [sidx])
        for dd in range(7):
            v_br[dd, sl] = plsc.load_gather(v_rois, [qv[dd], sidx])
            v_bgr[dd, sl] = plsc.load_gather(v_gtc, [qv[dd], asgs])
        v_gidx[sl] = sidx + b * R

    pltpu.async_copy(feat_hbm.at[v_gidx], v_bf, sem).wait()

    cp_out = [
        pltpu.async_copy(v_br, o_br.at[:, b], sem),
        pltpu.async_copy(v_bgr, o_bgr.at[:, b], sem),
        pltpu.async_copy(v_bi, o_bi.at[b], sem),
        pltpu.async_copy(v_bs, o_bs.at[b], sem),
        pltpu.async_copy(v_bl, o_bl.at[b], sem),
        pltpu.async_copy(v_bf, o_bf.at[b], sem),
        pltpu.async_copy(v_rvm, o_rvm.at[b], sem),
        pltpu.async_copy(v_cls, o_cls.at[b], sem),
    ]
    for c in cp_out:
        c.wait()


@jax.jit
def kernel(rois, roi_scores, roi_labels, gt_boxes, roi_features):
    rois_q = jnp.transpose(rois, (2, 0, 1))
    gtc_q = jnp.transpose(gt_boxes, (0, 2, 1))
    feat2d = roi_features.reshape(B * R, F)
    perm = jnp.asarray(_PERM)

    f32, i32 = jnp.float32, jnp.int32
    out_type = (
        jax.ShapeDtypeStruct((7, B, ROI_PER_IMAGE), f32),
        jax.ShapeDtypeStruct((7, B, ROI_PER_IMAGE), f32),
        jax.ShapeDtypeStruct((B, ROI_PER_IMAGE), f32),
        jax.ShapeDtypeStruct((B, ROI_PER_IMAGE), f32),
        jax.ShapeDtypeStruct((B, ROI_PER_IMAGE), i32),
        jax.ShapeDtypeStruct((B, ROI_PER_IMAGE, F), f32),
        jax.ShapeDtypeStruct((B, ROI_PER_IMAGE), i32),
        jax.ShapeDtypeStruct((B, ROI_PER_IMAGE), f32),
    )
    scratch = [
        pltpu.VMEM((7, R), f32),
        pltpu.VMEM((8, N), f32),
        pltpu.VMEM((R,), i32),
        pltpu.VMEM((R,), f32),
        pltpu.VMEM((3 * R,), i32),
        pltpu.VMEM((3 * GROWS * 16,), f32),
        pltpu.VMEM((3 * GROWS * 16,), f32),
        pltpu.VMEM((GROWS * 16,), f32),
        pltpu.VMEM((GROWS,), i32),
        pltpu.VMEM((3 * R,), i32),
        pltpu.VMEM((R,), f32),
        pltpu.VMEM((R,), i32),
        pltpu.VMEM((ROI_PER_IMAGE,), i32),
        pltpu.VMEM((ROI_PER_IMAGE,), i32),
        pltpu.VMEM((7, ROI_PER_IMAGE), f32),
        pltpu.VMEM((7, ROI_PER_IMAGE), f32),
        pltpu.VMEM((ROI_PER_IMAGE,), f32),
        pltpu.VMEM((ROI_PER_IMAGE,), f32),
        pltpu.VMEM((ROI_PER_IMAGE,), i32),
        pltpu.VMEM((ROI_PER_IMAGE,), i32),
        pltpu.VMEM((ROI_PER_IMAGE,), f32),
        pltpu.VMEM((ROI_PER_IMAGE, F), f32),
        pltpu.SemaphoreType.DMA,
    ]
    mesh = plsc.VectorSubcoreMesh(core_axis_name="c", subcore_axis_name="s")
    brq, bgrq, bi, bs, bl, bf, rvm, cls = pl.kernel(
        _body, out_type=out_type, mesh=mesh, scratch_types=scratch,
        compiler_params=pltpu.CompilerParams(needs_layout_passes=False),
    )(rois_q, gtc_q, roi_labels, roi_scores, perm, feat2d)

    br = jnp.transpose(brq, (1, 2, 0))
    bgr = jnp.transpose(bgrq, (1, 2, 0))
    return br, bgr, bi, bs, bl, bf, rvm, cls


# speedup vs baseline: 1.1393x; 1.0559x over previous
import jax
import jax.numpy as jnp
import numpy as np
from jax import lax
from jax.experimental import pallas as pl
from jax.experimental.pallas import tpu as pltpu
from jax.experimental.pallas import tpu_sc as plsc

B, R, N, F = 32, 1024, 64, 128
ROI_PER_IMAGE = 128
FG_NUM = 64
HARD_NUM = 51
EASY_NUM = 13
REG_FG_THRESH = 0.55
CLS_FG_THRESH = 0.75
CLS_BG_THRESH = 0.25
CLS_BG_THRESH_LO = 0.1

NVEC_R = R // 16
NVEC_S = ROI_PER_IMAGE // 16
NVEC_G = N // 16
GROWS = 80


def _host_perms() -> np.ndarray:
    u = np.uint32

    def tf(k, n):
        i = np.arange(n, dtype=np.uint64)
        x0 = (i >> np.uint64(32)).astype(u)
        x1 = (i & np.uint64(0xFFFFFFFF)).astype(u)
        rot = ((13, 15, 26, 6), (17, 29, 16, 24))
        ks = (u(k[0]), u(k[1]), u(k[0]) ^ u(k[1]) ^ u(0x1BD11BDA))
        sched = ((ks[1], ks[2]), (ks[2], ks[0]), (ks[0], ks[1]),
                 (ks[1], ks[2]), (ks[2], ks[0]))
        with np.errstate(over="ignore"):
            x0 = x0 + ks[0]
            x1 = x1 + ks[1]
            for gi in range(5):
                for rr in rot[gi % 2]:
                    x0 = x0 + x1
                    x1 = (x1 << u(rr)) | (x1 >> u(32 - rr))
                    x1 = x1 ^ x0
                x0 = x0 + sched[gi][0]
                x1 = x1 + sched[gi][1] + u(gi + 1)
        return x0, x1

    def split(k, n):
        x0, x1 = tf(k, n)
        return list(zip(x0, x1))

    def uniform01(k, n):
        x0, x1 = tf(k, n)
        bits = x0 ^ x1
        return ((bits >> u(9)) | u(0x3F800000)).view(np.float32) - np.float32(1.0)

    r = np.stack([np.stack([uniform01(kj, R) for kj in split(kb, 3)])
                  for kb in split((u(0), u(1)), B)])
    return np.argsort(-r, axis=-1, kind="stable").astype(np.int32)


_PERM = _host_perms().reshape(B, 3 * R)


def _body(rois_hbm, gtc_hbm, lab_hbm, sco_hbm, perm_hbm, feat_hbm,
          o_br, o_bgr, o_bi, o_bs, o_bl, o_bf, o_rvm, o_cls,
          v_rois, v_gtc, v_lab, v_sco, v_perm,
          v_bmin, v_bmax, v_vb, v_gorig, v_rcls,
          v_mo, v_asg, v_idx, v_gidx,
          v_br, v_bgr, v_bi, v_bs, v_bl, v_rvm, v_cls, v_bf,
          sem):
    b = lax.axis_index("c") * 16 + lax.axis_index("s")

    cp_in = [
        pltpu.async_copy(rois_hbm.at[:, b], v_rois, sem),
        pltpu.async_copy(gtc_hbm.at[b], v_gtc, sem),
        pltpu.async_copy(lab_hbm.at[b], v_lab, sem),
        pltpu.async_copy(sco_hbm.at[b], v_sco, sem),
        pltpu.async_copy(perm_hbm.at[b], v_perm, sem),
    ]
    for c in cp_in:
        c.wait()

    lane = lax.iota(jnp.int32, 16)
    zero16i = jnp.full((16,), 0, jnp.int32)
    qv = [zero16i + q for q in range(8)]

    glab = [v_gtc[7, pl.ds(ch * 16, 16)].astype(jnp.int32)
            for ch in range(NVEC_G)]
    gcnt = [jnp.full((16,), 0, jnp.int32) for _ in range(3)]
    for ch in range(NVEC_G):
        for c in range(3):
            gcnt[c] = gcnt[c] + plsc.all_reduce_population_count(glab[ch] == c + 1)
    a0 = (gcnt[0] + 3) & -4
    a1 = (gcnt[1] + 3) & -4
    gstart = [zero16i, a0, a0 + a1]

    grank = [jnp.full((16,), 0, jnp.int32) for _ in range(3)]
    for ch in range(NVEC_G):
        o = ch * 16
        labv = glab[ch]
        dest = zero16i
        for c in range(3):
            m = labv == c + 1
            pc = plsc.cumsum(m.astype(jnp.int32))
            dest = jnp.where(m, gstart[c] + grank[c] + pc - 1, dest)
            grank[c] = grank[c] + plsc.all_reduce_population_count(m)
        plsc.store_scatter(v_gorig, [dest], lane + o)
        c0 = v_gtc[0, pl.ds(o, 16)]
        c1 = v_gtc[1, pl.ds(o, 16)]
        c2 = v_gtc[2, pl.ds(o, 16)]
        s0 = v_gtc[3, pl.ds(o, 16)]
        s1 = v_gtc[4, pl.ds(o, 16)]
        s2 = v_gtc[5, pl.ds(o, 16)]
        rows = (c0 - s0 * 0.5, c1 - s1 * 0.5, c2 - s2 * 0.5,
                c0 + s0 * 0.5, c1 + s1 * 0.5, c2 + s2 * 0.5,
                (s0 * s1) * s2)
        refs = (v_bmin, v_bmin, v_bmin, v_bmax, v_bmax, v_bmax, v_vb)
        offs = (0, GROWS * 16, 2 * GROWS * 16, 0, GROWS * 16, 2 * GROWS * 16, 0)
        dest16 = dest * 16
        for l in range(16):
            idx = dest16 + l
            for rowv, ref, qoff in zip(rows, refs, offs):
                plsc.store_scatter(ref, [idx + qoff], rowv)

    def rpass(v, carry):
        labv = v_lab[pl.ds(v * 16, 16)]
        idxvec = lane + v * 16
        out = []
        for c in range(3):
            m = labv == c + 1
            pc = plsc.cumsum(m.astype(jnp.int32))
            slot = carry[c] + pc - 1
            plsc.store_scatter(v_rcls, [slot + c * R], idxvec, mask=m)
            out.append(carry[c] + plsc.all_reduce_population_count(m))
        return tuple(out)

    rcnt = lax.fori_loop(0, NVEC_R, rpass, (zero16i, zero16i, zero16i))

    for c in range(3):
        r_lo = c * R
        r_n = jnp.max(rcnt[c])
        g_lo = jnp.max(gstart[c])
        g_hi = g_lo + jnp.max(gcnt[c])
        nchunk = (r_n + 15) // 16

        def chunk_body(j, _, r_lo=r_lo, r_n=r_n, g_lo=g_lo, g_hi=g_hi):
            idxv = v_rcls[pl.ds(r_lo + j * 16, 16)] & (R - 1)
            lanemask = lane < (r_n - j * 16)
            cx = plsc.load_gather(v_rois, [qv[0], idxv])
            cy = plsc.load_gather(v_rois, [qv[1], idxv])
            cz = plsc.load_gather(v_rois, [qv[2], idxv])
            dx = plsc.load_gather(v_rois, [qv[3], idxv])
            dy = plsc.load_gather(v_rois, [qv[4], idxv])
            dz = plsc.load_gather(v_rois, [qv[5], idxv])
            hx = dx * 0.5
            hy = dy * 0.5
            hz = dz * 0.5
            ax0 = cx - hx
            ax1 = cx + hx
            ay0 = cy - hy
            ay1 = cy + hy
            az0 = cz - hz
            az1 = cz + hz
            va = dx * dy * dz

            def inner(g, carry):
                bn, bd, bidx = carry
                go = g * 16
                ix = jnp.maximum(jnp.minimum(ax1, v_bmax[pl.ds(go, 16)])
                                 - jnp.maximum(ax0, v_bmin[pl.ds(go, 16)]), 0.0)
                iy = jnp.maximum(jnp.minimum(ay1, v_bmax[pl.ds(GROWS * 16 + go, 16)])
                                 - jnp.maximum(ay0, v_bmin[pl.ds(GROWS * 16 + go, 16)]), 0.0)
                iz = jnp.maximum(jnp.minimum(az1, v_bmax[pl.ds(2 * GROWS * 16 + go, 16)])
                                 - jnp.maximum(az0, v_bmin[pl.ds(2 * GROWS * 16 + go, 16)]), 0.0)
                iv = (ix * iy) * iz
                den = jnp.maximum(va + v_vb[pl.ds(go, 16)] - iv, 1e-6)
                better = iv * bd > bn * den
                gvec = zero16i + g
                bn = jnp.where(better, iv, bn)
                bd = jnp.where(better, den, bd)
                bidx = jnp.where(better, gvec, bidx)
                return bn, bd, bidx

            init = (jnp.full((16,), 0.0, jnp.float32),
                    jnp.full((16,), 1.0, jnp.float32),
                    jnp.full((16,), -1, jnp.int32))
            bn, bd, bidx = lax.fori_loop(g_lo, g_hi, inner, init)
            mo = bn / bd
            asg = plsc.load_gather(v_gorig, [jnp.maximum(bidx, 0)])
            asg = jnp.where(bidx < 0, 0, asg)
            plsc.store_scatter(v_mo, [idxv], mo, mask=lanemask)
            plsc.store_scatter(v_asg, [idxv], asg, mask=lanemask)
            return ()

        lax.fori_loop(0, nchunk, chunk_body, ())

    thresholds = [
        lambda v: v >= REG_FG_THRESH,
        lambda v: jnp.logical_and(v < REG_FG_THRESH, v >= CLS_BG_THRESH_LO),
        lambda v: v < CLS_BG_THRESH_LO,
    ]
    pool_off = (0, FG_NUM, FG_NUM + HARD_NUM)
    pool_k = (FG_NUM, HARD_NUM, EASY_NUM)

    def pass_a(v, carry):
        out = []
        for pool in range(3):
            pidx = v_perm[pl.ds(pool * R + v * 16, 16)]
            vals = plsc.load_gather(v_mo, [pidx])
            m = thresholds[pool](vals)
            pc = plsc.cumsum(m.astype(jnp.int32))
            slot = carry[pool] + pc - 1
            ok = jnp.logical_and(m, slot < pool_k[pool])
            plsc.store_scatter(v_idx, [slot + pool_off[pool]], pidx, mask=ok)
            out.append(carry[pool] + plsc.all_reduce_population_count(m))
        return tuple(out)

    cnt3 = lax.fori_loop(0, NVEC_R, pass_a, (zero16i, zero16i, zero16i))

    for pool in range(3):
        off, k, cnt = pool_off[pool], pool_k[pool], cnt3[pool]
        mask_fn = thresholds[pool]

        def cond_b(st, k=k):
            v, cnt = st
            return jnp.logical_and(v < NVEC_R, jnp.max(cnt) < k)

        def pass_b(st, mask_fn=mask_fn, off=off, k=k):
            v, cnt = st
            pidx = lane + v * 16
            vals = v_mo[pl.ds(v * 16, 16)]
            m = jnp.logical_not(mask_fn(vals))
            pc = plsc.cumsum(m.astype(jnp.int32))
            slot = cnt + pc - 1
            ok = jnp.logical_and(m, slot < k)
            plsc.store_scatter(v_idx, [slot + off], pidx, mask=ok)
            return v + 1, cnt + plsc.all_reduce_population_count(m)

        lax.while_loop(cond_b, pass_b, (0, cnt))

    for s in range(NVEC_S):
        sl = pl.ds(s * 16, 16)
        sidx = v_idx[sl]
        iou_s = plsc.load_gather(v_mo, [sidx])
        v_bi[sl] = iou_s
        v_rvm[sl] = (iou_s > REG_FG_THRESH).astype(jnp.int32)
        fgm = iou_s > CLS_FG_THRESH
        bgm = iou_s < CLS_BG_THRESH
        interval = jnp.logical_and(jnp.logical_not(fgm), jnp.logical_not(bgm))
        v_cls[sl] = jnp.where(interval, (iou_s - CLS_BG_THRESH) * 2.0,
                              jnp.where(fgm, 1.0, 0.0))
        v_bl[sl] = plsc.load_gather(v_lab, [sidx])
        v_bs[sl] = plsc.load_gather(v_sco, [sidx])
        asgs = plsc.load_gather(v_asg, [sidx])
        for dd in range(7):
            v_br[dd, sl] = plsc.load_gather(v_rois, [qv[dd], sidx])
            v_bgr[dd, sl] = plsc.load_gather(v_gtc, [qv[dd], asgs])
        v_gidx[sl] = sidx + b * R

    pltpu.async_copy(feat_hbm.at[v_gidx], v_bf, sem).wait()

    cp_out = [
        pltpu.async_copy(v_br, o_br.at[:, b], sem),
        pltpu.async_copy(v_bgr, o_bgr.at[:, b], sem),
        pltpu.async_copy(v_bi, o_bi.at[b], sem),
        pltpu.async_copy(v_bs, o_bs.at[b], sem),
        pltpu.async_copy(v_bl, o_bl.at[b], sem),
        pltpu.async_copy(v_bf, o_bf.at[b], sem),
        pltpu.async_copy(v_rvm, o_rvm.at[b], sem),
        pltpu.async_copy(v_cls, o_cls.at[b], sem),
    ]
    for c in cp_out:
        c.wait()


@jax.jit
def kernel(rois, roi_scores, roi_labels, gt_boxes, roi_features):
    rois_q = jnp.transpose(rois, (2, 0, 1))
    gtc_q = jnp.transpose(gt_boxes, (0, 2, 1))
    feat2d = roi_features.reshape(B * R, F)
    perm = jnp.asarray(_PERM)

    f32, i32 = jnp.float32, jnp.int32
    out_type = (
        jax.ShapeDtypeStruct((7, B, ROI_PER_IMAGE), f32),
        jax.ShapeDtypeStruct((7, B, ROI_PER_IMAGE), f32),
        jax.ShapeDtypeStruct((B, ROI_PER_IMAGE), f32),
        jax.ShapeDtypeStruct((B, ROI_PER_IMAGE), f32),
        jax.ShapeDtypeStruct((B, ROI_PER_IMAGE), i32),
        jax.ShapeDtypeStruct((B, ROI_PER_IMAGE, F), f32),
        jax.ShapeDtypeStruct((B, ROI_PER_IMAGE), i32),
        jax.ShapeDtypeStruct((B, ROI_PER_IMAGE), f32),
    )
    scratch = [
        pltpu.VMEM((7, R), f32),
        pltpu.VMEM((8, N), f32),
        pltpu.VMEM((R,), i32),
        pltpu.VMEM((R,), f32),
        pltpu.VMEM((3 * R,), i32),
        pltpu.VMEM((3 * GROWS * 16,), f32),
        pltpu.VMEM((3 * GROWS * 16,), f32),
        pltpu.VMEM((GROWS * 16,), f32),
        pltpu.VMEM((GROWS,), i32),
        pltpu.VMEM((3 * R,), i32),
        pltpu.VMEM((R,), f32),
        pltpu.VMEM((R,), i32),
        pltpu.VMEM((ROI_PER_IMAGE,), i32),
        pltpu.VMEM((ROI_PER_IMAGE,), i32),
        pltpu.VMEM((7, ROI_PER_IMAGE), f32),
        pltpu.VMEM((7, ROI_PER_IMAGE), f32),
        pltpu.VMEM((ROI_PER_IMAGE,), f32),
        pltpu.VMEM((ROI_PER_IMAGE,), f32),
        pltpu.VMEM((ROI_PER_IMAGE,), i32),
        pltpu.VMEM((ROI_PER_IMAGE,), i32),
        pltpu.VMEM((ROI_PER_IMAGE,), f32),
        pltpu.VMEM((ROI_PER_IMAGE, F), f32),
        pltpu.SemaphoreType.DMA,
    ]
    mesh = plsc.VectorSubcoreMesh(core_axis_name="c", subcore_axis_name="s")
    brq, bgrq, bi, bs, bl, bf, rvm, cls = pl.kernel(
        _body, out_type=out_type, mesh=mesh, scratch_types=scratch,
        compiler_params=pltpu.CompilerParams(needs_layout_passes=False),
    )(rois_q, gtc_q, roi_labels, roi_scores, perm, feat2d)

    br = jnp.transpose(brq, (1, 2, 0))
    bgr = jnp.transpose(bgrq, (1, 2, 0))
    return br, bgr, bi, bs, bl, bf, rvm, cls


# speedup vs baseline: 1.1616x; 1.0196x over previous
import jax
import jax.numpy as jnp
import numpy as np
from jax import lax
from jax.experimental import pallas as pl
from jax.experimental.pallas import tpu as pltpu
from jax.experimental.pallas import tpu_sc as plsc

B, R, N, F = 32, 1024, 64, 128
ROI_PER_IMAGE = 128
FG_NUM = 64
HARD_NUM = 51
EASY_NUM = 13
REG_FG_THRESH = 0.55
CLS_FG_THRESH = 0.75
CLS_BG_THRESH = 0.25
CLS_BG_THRESH_LO = 0.1

NVEC_R = R // 16
NVEC_S = ROI_PER_IMAGE // 16
NVEC_G = N // 16
GROWS = 80


def _host_perms() -> np.ndarray:
    u = np.uint32

    def tf(k, n):
        i = np.arange(n, dtype=np.uint64)
        x0 = (i >> np.uint64(32)).astype(u)
        x1 = (i & np.uint64(0xFFFFFFFF)).astype(u)
        rot = ((13, 15, 26, 6), (17, 29, 16, 24))
        ks = (u(k[0]), u(k[1]), u(k[0]) ^ u(k[1]) ^ u(0x1BD11BDA))
        sched = ((ks[1], ks[2]), (ks[2], ks[0]), (ks[0], ks[1]),
                 (ks[1], ks[2]), (ks[2], ks[0]))
        with np.errstate(over="ignore"):
            x0 = x0 + ks[0]
            x1 = x1 + ks[1]
            for gi in range(5):
                for rr in rot[gi % 2]:
                    x0 = x0 + x1
                    x1 = (x1 << u(rr)) | (x1 >> u(32 - rr))
                    x1 = x1 ^ x0
                x0 = x0 + sched[gi][0]
                x1 = x1 + sched[gi][1] + u(gi + 1)
        return x0, x1

    def split(k, n):
        x0, x1 = tf(k, n)
        return list(zip(x0, x1))

    def uniform01(k, n):
        x0, x1 = tf(k, n)
        bits = x0 ^ x1
        return ((bits >> u(9)) | u(0x3F800000)).view(np.float32) - np.float32(1.0)

    r = np.stack([np.stack([uniform01(kj, R) for kj in split(kb, 3)])
                  for kb in split((u(0), u(1)), B)])
    return np.argsort(-r, axis=-1, kind="stable").astype(np.int32)


_PERM = _host_perms().reshape(B, 3 * R)


def _body(rois_hbm, gtc_hbm, lab_hbm, sco_hbm, perm_hbm, feat_hbm,
          o_br, o_bgr, o_bi, o_bs, o_bl, o_bf, o_rvm, o_cls,
          v_rois, v_gtc, v_lab, v_sco, v_perm,
          v_bmin, v_bmax, v_vb, v_gorig, v_rcls,
          v_mo, v_asg, v_idx, v_gidx,
          v_br, v_bgr, v_bi, v_bs, v_bl, v_rvm, v_cls, v_bf,
          sem):
    b = lax.axis_index("c") * 16 + lax.axis_index("s")

    cp_in = [
        pltpu.async_copy(rois_hbm.at[:, b], v_rois, sem),
        pltpu.async_copy(gtc_hbm.at[b], v_gtc, sem),
        pltpu.async_copy(lab_hbm.at[b], v_lab, sem),
        pltpu.async_copy(sco_hbm.at[b], v_sco, sem),
        pltpu.async_copy(perm_hbm.at[b], v_perm, sem),
    ]
    for c in cp_in:
        c.wait()

    lane = lax.iota(jnp.int32, 16)
    zero16i = jnp.full((16,), 0, jnp.int32)
    qv = [zero16i + q for q in range(8)]

    glab = [v_gtc[7, pl.ds(ch * 16, 16)].astype(jnp.int32)
            for ch in range(NVEC_G)]
    gcnt = [jnp.full((16,), 0, jnp.int32) for _ in range(3)]
    for ch in range(NVEC_G):
        for c in range(3):
            gcnt[c] = gcnt[c] + plsc.all_reduce_population_count(glab[ch] == c + 1)
    a0 = (gcnt[0] + 3) & -4
    a1 = (gcnt[1] + 3) & -4
    gstart = [zero16i, a0, a0 + a1]

    grank = [jnp.full((16,), 0, jnp.int32) for _ in range(3)]
    for ch in range(NVEC_G):
        o = ch * 16
        labv = glab[ch]
        dest = zero16i
        for c in range(3):
            m = labv == c + 1
            pc = plsc.cumsum(m.astype(jnp.int32))
            dest = jnp.where(m, gstart[c] + grank[c] + pc - 1, dest)
            grank[c] = grank[c] + plsc.all_reduce_population_count(m)
        plsc.store_scatter(v_gorig, [dest], lane + o)
        c0 = v_gtc[0, pl.ds(o, 16)]
        c1 = v_gtc[1, pl.ds(o, 16)]
        c2 = v_gtc[2, pl.ds(o, 16)]
        s0 = v_gtc[3, pl.ds(o, 16)]
        s1 = v_gtc[4, pl.ds(o, 16)]
        s2 = v_gtc[5, pl.ds(o, 16)]
        rows = (c0 - s0 * 0.5, c1 - s1 * 0.5, c2 - s2 * 0.5,
                c0 + s0 * 0.5, c1 + s1 * 0.5, c2 + s2 * 0.5,
                (s0 * s1) * s2)
        refs = (v_bmin, v_bmin, v_bmin, v_bmax, v_bmax, v_bmax, v_vb)
        offs = (0, GROWS * 16, 2 * GROWS * 16, 0, GROWS * 16, 2 * GROWS * 16, 0)
        dest16 = dest * 16

        def lane_scatter(l, _, rows=rows, dest16=dest16):
            idx = dest16 + l
            for rowv, ref, qoff in zip(rows, refs, offs):
                plsc.store_scatter(ref, [idx + qoff], rowv)
            return ()

        lax.fori_loop(0, 16, lane_scatter, ())

    def rpass(v, carry):
        labv = v_lab[pl.ds(v * 16, 16)]
        idxvec = lane + v * 16
        out = []
        for c in range(3):
            m = labv == c + 1
            pc = plsc.cumsum(m.astype(jnp.int32))
            slot = carry[c] + pc - 1
            plsc.store_scatter(v_rcls, [slot + c * R], idxvec, mask=m)
            out.append(carry[c] + plsc.all_reduce_population_count(m))
        return tuple(out)

    rcnt = lax.fori_loop(0, NVEC_R, rpass, (zero16i, zero16i, zero16i))

    def class_sweep(c, _):
        r_lo = c * R
        r_n = jnp.max(jnp.where(c == 0, rcnt[0],
                                jnp.where(c == 1, rcnt[1], rcnt[2])))
        g_lo = jnp.max(jnp.where(c == 0, gstart[0],
                                 jnp.where(c == 1, gstart[1], gstart[2])))
        g_hi = g_lo + jnp.max(jnp.where(c == 0, gcnt[0],
                                        jnp.where(c == 1, gcnt[1], gcnt[2])))
        nchunk = (r_n + 15) // 16

        def chunk_body(j, _, r_lo=r_lo, r_n=r_n, g_lo=g_lo, g_hi=g_hi):
            idxv = v_rcls[pl.ds(r_lo + j * 16, 16)] & (R - 1)
            lanemask = lane < (r_n - j * 16)
            cx = plsc.load_gather(v_rois, [qv[0], idxv])
            cy = plsc.load_gather(v_rois, [qv[1], idxv])
            cz = plsc.load_gather(v_rois, [qv[2], idxv])
            dx = plsc.load_gather(v_rois, [qv[3], idxv])
            dy = plsc.load_gather(v_rois, [qv[4], idxv])
            dz = plsc.load_gather(v_rois, [qv[5], idxv])
            hx = dx * 0.5
            hy = dy * 0.5
            hz = dz * 0.5
            ax0 = cx - hx
            ax1 = cx + hx
            ay0 = cy - hy
            ay1 = cy + hy
            az0 = cz - hz
            az1 = cz + hz
            va = dx * dy * dz

            def inner(g, carry):
                bn, bd, bidx = carry
                go = g * 16
                ix = jnp.maximum(jnp.minimum(ax1, v_bmax[pl.ds(go, 16)])
                                 - jnp.maximum(ax0, v_bmin[pl.ds(go, 16)]), 0.0)
                iy = jnp.maximum(jnp.minimum(ay1, v_bmax[pl.ds(GROWS * 16 + go, 16)])
                                 - jnp.maximum(ay0, v_bmin[pl.ds(GROWS * 16 + go, 16)]), 0.0)
                iz = jnp.maximum(jnp.minimum(az1, v_bmax[pl.ds(2 * GROWS * 16 + go, 16)])
                                 - jnp.maximum(az0, v_bmin[pl.ds(2 * GROWS * 16 + go, 16)]), 0.0)
                iv = (ix * iy) * iz
                den = jnp.maximum(va + v_vb[pl.ds(go, 16)] - iv, 1e-6)
                better = iv * bd > bn * den
                gvec = zero16i + g
                bn = jnp.where(better, iv, bn)
                bd = jnp.where(better, den, bd)
                bidx = jnp.where(better, gvec, bidx)
                return bn, bd, bidx

            init = (jnp.full((16,), 0.0, jnp.float32),
                    jnp.full((16,), 1.0, jnp.float32),
                    jnp.full((16,), -1, jnp.int32))
            bn, bd, bidx = lax.fori_loop(g_lo, g_hi, inner, init)
            mo = bn / bd
            asg = plsc.load_gather(v_gorig, [jnp.maximum(bidx, 0)])
            asg = jnp.where(bidx < 0, 0, asg)
            plsc.store_scatter(v_mo, [idxv], mo, mask=lanemask)
            plsc.store_scatter(v_asg, [idxv], asg, mask=lanemask)
            return ()

        lax.fori_loop(0, nchunk, chunk_body, ())
        return ()

    lax.fori_loop(0, 3, class_sweep, ())

    thresholds = [
        lambda v: v >= REG_FG_THRESH,
        lambda v: jnp.logical_and(v < REG_FG_THRESH, v >= CLS_BG_THRESH_LO),
        lambda v: v < CLS_BG_THRESH_LO,
    ]
    pool_off = (0, FG_NUM, FG_NUM + HARD_NUM)
    pool_k = (FG_NUM, HARD_NUM, EASY_NUM)

    def pass_a(v, carry):
        out = []
        for pool in range(3):
            pidx = v_perm[pl.ds(pool * R + v * 16, 16)]
            vals = plsc.load_gather(v_mo, [pidx])
            m = thresholds[pool](vals)
            pc = plsc.cumsum(m.astype(jnp.int32))
            slot = carry[pool] + pc - 1
            ok = jnp.logical_and(m, slot < pool_k[pool])
            plsc.store_scatter(v_idx, [slot + pool_off[pool]], pidx, mask=ok)
            out.append(carry[pool] + plsc.all_reduce_population_count(m))
        return tuple(out)

    cnt3 = lax.fori_loop(0, NVEC_R, pass_a, (zero16i, zero16i, zero16i))

    for pool in range(3):
        off, k, cnt = pool_off[pool], pool_k[pool], cnt3[pool]
        mask_fn = thresholds[pool]

        def cond_b(st, k=k):
            v, cnt = st
            return jnp.logical_and(v < NVEC_R, jnp.max(cnt) < k)

        def pass_b(st, mask_fn=mask_fn, off=off, k=k):
            v, cnt = st
            pidx = lane + v * 16
            vals = v_mo[pl.ds(v * 16, 16)]
            m = jnp.logical_not(mask_fn(vals))
            pc = plsc.cumsum(m.astype(jnp.int32))
            slot = cnt + pc - 1
            ok = jnp.logical_and(m, slot < k)
            plsc.store_scatter(v_idx, [slot + off], pidx, mask=ok)
            return v + 1, cnt + plsc.all_reduce_population_count(m)

        lax.while_loop(cond_b, pass_b, (0, cnt))

    for s in range(NVEC_S):
        sl = pl.ds(s * 16, 16)
        sidx = v_idx[sl]
        iou_s = plsc.load_gather(v_mo, [sidx])
        v_bi[sl] = iou_s
        v_rvm[sl] = (iou_s > REG_FG_THRESH).astype(jnp.int32)
        fgm = iou_s > CLS_FG_THRESH
        bgm = iou_s < CLS_BG_THRESH
        interval = jnp.logical_and(jnp.logical_not(fgm), jnp.logical_not(bgm))
        v_cls[sl] = jnp.where(interval, (iou_s - CLS_BG_THRESH) * 2.0,
                              jnp.where(fgm, 1.0, 0.0))
        v_bl[sl] = plsc.load_gather(v_lab, [sidx])
        v_bs[sl] = plsc.load_gather(v_sco, [sidx])
        asgs = plsc.load_gather(v_asg, [sidx])
        for dd in range(7):
            v_br[dd, sl] = plsc.load_gather(v_rois, [qv[dd], sidx])
            v_bgr[dd, sl] = plsc.load_gather(v_gtc, [qv[dd], asgs])
        v_gidx[sl] = sidx + b * R

    pltpu.async_copy(feat_hbm.at[v_gidx], v_bf, sem).wait()

    cp_out = [
        pltpu.async_copy(v_br, o_br.at[:, b], sem),
        pltpu.async_copy(v_bgr, o_bgr.at[:, b], sem),
        pltpu.async_copy(v_bi, o_bi.at[b], sem),
        pltpu.async_copy(v_bs, o_bs.at[b], sem),
        pltpu.async_copy(v_bl, o_bl.at[b], sem),
        pltpu.async_copy(v_bf, o_bf.at[b], sem),
        pltpu.async_copy(v_rvm, o_rvm.at[b], sem),
        pltpu.async_copy(v_cls, o_cls.at[b], sem),
    ]
    for c in cp_out:
        c.wait()


@jax.jit
def kernel(rois, roi_scores, roi_labels, gt_boxes, roi_features):
    rois_q = jnp.transpose(rois, (2, 0, 1))
    gtc_q = jnp.transpose(gt_boxes, (0, 2, 1))
    feat2d = roi_features.reshape(B * R, F)
    perm = jnp.asarray(_PERM)

    f32, i32 = jnp.float32, jnp.int32
    out_type = (
        jax.ShapeDtypeStruct((7, B, ROI_PER_IMAGE), f32),
        jax.ShapeDtypeStruct((7, B, ROI_PER_IMAGE), f32),
        jax.ShapeDtypeStruct((B, ROI_PER_IMAGE), f32),
        jax.ShapeDtypeStruct((B, ROI_PER_IMAGE), f32),
        jax.ShapeDtypeStruct((B, ROI_PER_IMAGE), i32),
        jax.ShapeDtypeStruct((B, ROI_PER_IMAGE, F), f32),
        jax.ShapeDtypeStruct((B, ROI_PER_IMAGE), i32),
        jax.ShapeDtypeStruct((B, ROI_PER_IMAGE), f32),
    )
    scratch = [
        pltpu.VMEM((7, R), f32),
        pltpu.VMEM((8, N), f32),
        pltpu.VMEM((R,), i32),
        pltpu.VMEM((R,), f32),
        pltpu.VMEM((3 * R,), i32),
        pltpu.VMEM((3 * GROWS * 16,), f32),
        pltpu.VMEM((3 * GROWS * 16,), f32),
        pltpu.VMEM((GROWS * 16,), f32),
        pltpu.VMEM((GROWS,), i32),
        pltpu.VMEM((3 * R,), i32),
        pltpu.VMEM((R,), f32),
        pltpu.VMEM((R,), i32),
        pltpu.VMEM((ROI_PER_IMAGE,), i32),
        pltpu.VMEM((ROI_PER_IMAGE,), i32),
        pltpu.VMEM((7, ROI_PER_IMAGE), f32),
        pltpu.VMEM((7, ROI_PER_IMAGE), f32),
        pltpu.VMEM((ROI_PER_IMAGE,), f32),
        pltpu.VMEM((ROI_PER_IMAGE,), f32),
        pltpu.VMEM((ROI_PER_IMAGE,), i32),
        pltpu.VMEM((ROI_PER_IMAGE,), i32),
        pltpu.VMEM((ROI_PER_IMAGE,), f32),
        pltpu.VMEM((ROI_PER_IMAGE, F), f32),
        pltpu.SemaphoreType.DMA,
    ]
    mesh = plsc.VectorSubcoreMesh(core_axis_name="c", subcore_axis_name="s")
    brq, bgrq, bi, bs, bl, bf, rvm, cls = pl.kernel(
        _body, out_type=out_type, mesh=mesh, scratch_types=scratch,
        compiler_params=pltpu.CompilerParams(needs_layout_passes=False),
    )(rois_q, gtc_q, roi_labels, roi_scores, perm, feat2d)

    br = jnp.transpose(brq, (1, 2, 0))
    bgr = jnp.transpose(bgrq, (1, 2, 0))
    return br, bgr, bi, bs, bl, bf, rvm, cls
